# baseline (speedup 1.0000x reference)
import functools

import jax
import jax.numpy as jnp
from jax import lax
from jax.experimental import pallas as pl
from jax.experimental.pallas import tpu as pltpu
from jax.experimental.pallas import tpu_sc as plsc

TOKENS = 16384
D2 = 4096
HALF = D2 // 2
G = 8
L = 16
NC, NS = 2, 16
NW = NC * NS
ROWS_PER_W = TOKENS // NW
R = 8
NCHUNK = ROWS_PER_W // R
NSTEP = NCHUNK // 2
MAGIC = 12582912.0
WROW = HALF // 4


def _sc_body(x_hbm, gpad_hbm, tabf_hbm, outq_hbm, outs_hbm,
             inb0, inb1, tabf_v, stag, pk0, pk1, sb0, sb1, gv,
             sem_i0, sem_i1, sem_q0, sem_q1, sem_s0, sem_s1):
    wid = lax.axis_index("s") * NC + lax.axis_index("c")
    base = wid * ROWS_PER_W

    pltpu.sync_copy(gpad_hbm, gv)
    pltpu.sync_copy(tabf_hbm, tabf_v)
    g = gv[...]

    inb = (inb0, inb1)
    pk = (pk0, pk1)
    sb = (sb0, sb1)
    sem_i = (sem_i0, sem_i1)
    sem_q = (sem_q0, sem_q1)
    sem_s = (sem_s0, sem_s1)

    def in_copy(c, par):
        return pltpu.make_async_copy(
            x_hbm.at[pl.ds(base + c * R, R)], inb[par], sem_i[par])

    def q_copy(c, par):
        return pltpu.make_async_copy(
            pk[par], outq_hbm.at[pl.ds(base + c * R, R)], sem_q[par])

    def s_copy(c, par):
        return pltpu.make_async_copy(
            sb[par], outs_hbm.at[pl.ds(base + c * R, R)], sem_s[par])

    in_copy(0, 0).start()
    in_copy(1, 1).start()

    lanes = lax.iota(jnp.int32, L)
    _dnums = lax.GatherDimensionNumbers(
        offset_dims=(), collapsed_slice_dims=(0,), start_index_map=(0,))

    def shuf(v, idx):
        return lax.gather(v, idx[:, None], _dnums, slice_sizes=(1,),
                          mode=lax.GatherScatterMode.PROMISE_IN_BOUNDS)

    def lane_max(v):
        for sh in (1, 2, 4, 8):
            v = jnp.maximum(v, shuf(v, lanes ^ sh))
        return v

    mloc = 4 * (lanes & 3)

    def process_chunk(c, par):
        src = inb[par]
        dstq = pk[par]
        dsts = sb[par]
        for r in range(R):
            row = base + c * R + r
            gid = jnp.int32(0)
            for k in range(G):
                gid = gid + (row >= g[k]).astype(jnp.int32)
            soff = gid * HALF

            def p1(j, mx):
                for u in range(4):
                    jj = 4 * j + u
                    lft = src[r, pl.ds(jj * L, L)]
                    rgt = src[r, pl.ds(HALF + jj * L, L)]
                    sc = tabf_v[pl.ds(soff + jj * L, L)]
                    sig = 1.0 / (1.0 + jnp.exp(-rgt))
                    o = (sig * rgt) * lft * sc
                    stag[pl.ds(jj * L, L)] = o
                    mx = jnp.maximum(mx, jnp.abs(o))
                return mx

            mx = lax.fori_loop(0, HALF // (4 * L), p1,
                               jnp.zeros((L,), jnp.float32))
            mx = lane_max(jnp.maximum(mx, 1e-10))
            dscv = 127.0 / mx
            dsts[r, :] = dscv

            def p2(j2, _):
                for u in range(2):
                    jj = 2 * j2 + u
                    e0 = jj * 64
                    cq = []
                    for k in range(4):
                        p = stag[pl.ds(e0 + k * L, L)]
                        v = jnp.clip(p * dscv, -128.0, 127.0)
                        cq.append(((v + MAGIC) - MAGIC).astype(jnp.int32))
                    ws = None
                    for b in range(4):
                        m = mloc + b
                        gk = [shuf(cq[k], m) for k in range(4)]
                        s = jnp.where(lanes < 8,
                                      jnp.where(lanes < 4, gk[0], gk[1]),
                                      jnp.where(lanes < 12, gk[2], gk[3]))
                        t = (s & 255) << (8 * b)
                        ws = t if ws is None else ws | t
                    dstq[r, pl.ds(jj * L, L)] = ws
                return 0

            lax.fori_loop(0, WROW // (2 * L), p2, 0)

    def step(s, _):
        for par in range(2):
            c = 2 * s + par
            in_copy(c, par).wait()

            @pl.when(s >= 1)
            def _():
                q_copy(c - 2, par).wait()
                s_copy(c - 2, par).wait()

            process_chunk(c, par)

            @pl.when(s < NSTEP - 1)
            def _():
                in_copy(c + 2, par).start()

            q_copy(c, par).start()
            s_copy(c, par).start()
        return 0

    lax.fori_loop(0, NSTEP, step, 0)

    for par in range(2):
        q_copy(NCHUNK - 2 + par, par).wait()
        s_copy(NCHUNK - 2 + par, par).wait()


@jax.jit
def _run_sc(x, gpad, tabf):
    mesh = plsc.VectorSubcoreMesh(core_axis_name="c", subcore_axis_name="s",
                                  num_cores=NC, num_subcores=NS)
    f = pl.kernel(
        _sc_body,
        out_type=[
            jax.ShapeDtypeStruct((TOKENS, WROW), jnp.int32),
            jax.ShapeDtypeStruct((TOKENS, L), jnp.float32),
        ],
        mesh=mesh,
        scratch_types=[
            pltpu.VMEM((R, D2), jnp.float32),
            pltpu.VMEM((R, D2), jnp.float32),
            pltpu.VMEM(((G + 1) * HALF,), jnp.float32),
            pltpu.VMEM((HALF,), jnp.float32),
            pltpu.VMEM((R, WROW), jnp.int32),
            pltpu.VMEM((R, WROW), jnp.int32),
            pltpu.VMEM((R, L), jnp.float32),
            pltpu.VMEM((R, L), jnp.float32),
            pltpu.VMEM((L,), jnp.int32),
            pltpu.SemaphoreType.DMA,
            pltpu.SemaphoreType.DMA,
            pltpu.SemaphoreType.DMA,
            pltpu.SemaphoreType.DMA,
            pltpu.SemaphoreType.DMA,
            pltpu.SemaphoreType.DMA,
        ],
    )
    q32, s16 = f(x, gpad, tabf)
    q = lax.bitcast_convert_type(q32, jnp.int8).reshape(TOKENS, HALF)
    return q, s16[:, 0]


def kernel(x, smooth_scales, group_index, quant_mode):
    gpad = jnp.concatenate(
        [group_index.astype(jnp.int32),
         jnp.full((L - G,), TOKENS, jnp.int32)])
    tabf = jnp.concatenate(
        [smooth_scales.astype(jnp.float32),
         jnp.ones((1, HALF), jnp.float32)], axis=0).reshape(-1)
    return _run_sc(x, gpad, tabf)


# speedup vs baseline: 21.3564x; 21.3564x over previous
import functools

import jax
import jax.numpy as jnp
from jax.experimental import pallas as pl
from jax.experimental.pallas import tpu as pltpu

TOKENS = 16384
D2 = 4096
HALF = D2 // 2
G = 8
BR = 1024


def _fused_body(gi_ref, x_ref, tab_ref, q_ref, s_ref):
    x = x_ref[...]
    left = x[:, :HALF]
    right = x[:, HALF:]
    sig = 0.5 * jnp.tanh(0.5 * right) + 0.5
    sw = (sig * right) * left

    row0 = pl.program_id(0) * BR
    rows = row0 + jax.lax.broadcasted_iota(jnp.int32, (BR, 1), 0)
    gid = jnp.zeros((BR, 1), jnp.int32)
    for i in range(G):
        gid += (rows >= gi_ref[i]).astype(jnp.int32)
    onehot = (gid == jax.lax.broadcasted_iota(jnp.int32, (BR, G + 1), 1))
    scale = jnp.dot(onehot.astype(jnp.float32), tab_ref[...],
                    preferred_element_type=jnp.float32)
    out = sw * scale

    y_max = jnp.max(jnp.abs(out), axis=1, keepdims=True)
    y_max = jnp.maximum(y_max, 1e-10)
    ds = 127.0 / y_max
    q_ref[...] = jnp.round(out * ds).astype(jnp.int8)
    s_ref[...] = ds


@jax.jit
def _run(x, table, group_index):
    grid = (TOKENS // BR,)
    q, s = pl.pallas_call(
        _fused_body,
        grid=grid,
        in_specs=[
            pl.BlockSpec(memory_space=pltpu.SMEM),
            pl.BlockSpec((BR, D2), lambda i: (i, 0)),
            pl.BlockSpec((G + 1, HALF), lambda i: (0, 0)),
        ],
        out_specs=[
            pl.BlockSpec((BR, HALF), lambda i: (i, 0)),
            pl.BlockSpec((BR, 1), lambda i: (i, 0)),
        ],
        out_shape=[
            jax.ShapeDtypeStruct((TOKENS, HALF), jnp.int8),
            jax.ShapeDtypeStruct((TOKENS, 1), jnp.float32),
        ],
        compiler_params=pltpu.CompilerParams(
            dimension_semantics=("arbitrary",),
        ),
    )(group_index, x, table)
    return q, jnp.squeeze(s, axis=-1)


def kernel(x, smooth_scales, group_index, quant_mode):
    table = jnp.concatenate(
        [smooth_scales.astype(jnp.float32),
         jnp.ones((1, HALF), jnp.float32)], axis=0)
    return _run(x, table, group_index)
